# baseline (speedup 1.0000x reference)
import functools

import jax
import jax.numpy as jnp
from jax.experimental import pallas as pl
from jax.experimental.pallas import tpu as pltpu

LN_EPS = 1e-5
DELTA_EPS = 1e-6

Bb = 128
Lc = 128
WIN = 4
JB = 8


def _fused_kernel(nc_total, seq_ref, eWT, w1T, b1c, w2T, b2c, lngc, lnbc,
                  gw1T, gb1c, gw2c, gb2c, rwT, rbc, owT, obc,
                  out_ref, h_s, g_s, rd_s, M_s):
    c = pl.program_id(1)
    V = eWT.shape[1]
    H = h_s.shape[0]
    n = seq_ref.shape[-1]

    seq = seq_ref[0, 0]
    iota_v = jax.lax.broadcasted_iota(jnp.int32, (V, n), 0)
    oh = (iota_v == seq).astype(jnp.float32)
    h0 = jnp.dot(eWT[...], oh, preferred_element_type=jnp.float32)
    ff1 = jnp.maximum(
        jnp.dot(w1T[...], h0, preferred_element_type=jnp.float32) + b1c[...], 0.0)
    h1 = jnp.dot(w2T[...], ff1, preferred_element_type=jnp.float32) + b2c[...] + h0
    mu = jnp.mean(h1, axis=0, keepdims=True)
    d = h1 - mu
    var = jnp.mean(d * d, axis=0, keepdims=True)
    hn = d * jax.lax.rsqrt(var + LN_EPS) * lngc[...] + lnbc[...]
    h_s[...] = hn

    g1 = jnp.maximum(
        jnp.dot(gw1T[...], hn, preferred_element_type=jnp.float32) + gb1c[...], 0.0)
    gp = jnp.sum(g1 * gw2c[...], axis=0, keepdims=True) + gb2c[...]
    gate = jax.nn.sigmoid(gp)
    lastpos = jax.lax.broadcasted_iota(jnp.int32, (1, n), 1) >= (n - Bb)
    is_last_chunk = (c == nc_total - 1)
    gate = jnp.where(jnp.logical_and(is_last_chunk, lastpos), 0.0, gate)
    g_s[...] = gate
    rd_s[...] = 1.0 / (jnp.sum(hn * hn, axis=0, keepdims=True) + DELTA_EPS)

    @pl.when(c == 0)
    def _():
        M_s[...] = jnp.zeros_like(M_s)

    def window(w, carry):
        base = w * (WIN * Bb)
        ks = [h_s[:, pl.ds(base + t * Bb, Bb)] for t in range(WIN)]
        gs = [g_s[:, pl.ds(base + t * Bb, Bb)] for t in range(WIN)]
        rds = [rd_s[:, pl.ds(base + t * Bb, Bb)] for t in range(WIN)]

        acc = [None] * WIN
        for jb in range(0, H, JB):
            blk = M_s[jb:jb + JB]
            for t in range(WIN):
                part = jnp.sum(blk * ks[t][jb:jb + JB, None, :], axis=0)
                acc[t] = part if jb == 0 else acc[t] + part

        us = [None] * WIN
        for t in range(WIN):
            a_t = acc[t]
            for s in range(t):
                d_st = jnp.sum(ks[s] * ks[t], axis=0, keepdims=True)
                a_t = a_t + d_st * us[s]
            us[t] = gs[t] * (ks[t] - a_t * rds[t])

        for jb in range(0, H, JB):
            blk = M_s[jb:jb + JB]
            for t in range(WIN):
                blk = blk + ks[t][jb:jb + JB, None, :] * us[t][None, :, :]
            M_s[jb:jb + JB] = blk
        return carry

    jax.lax.fori_loop(0, Lc // WIN, window, 0)

    @pl.when(c == nc_total - 1)
    def _():
        q = h_s[:, (Lc - 1) * Bb:]
        ctx = jnp.sum(M_s[...] * q[:, None, :], axis=0)
        r = jnp.dot(rwT[...], ctx, preferred_element_type=jnp.float32) + rbc[...]
        out_ref[...] = jnp.dot(owT[...], r, preferred_element_type=jnp.float32) + obc[...]


def kernel(seq, embed_W, ff_w1, ff_b1, ff_w2, ff_b2, ln_g, ln_b,
           gate_w1, gate_b1, gate_w2, gate_b2, read_w, read_b, out_w, out_b):
    B, L = seq.shape
    V, H = embed_W.shape
    nb, nc = B // Bb, L // Lc
    N = Lc * Bb

    seq4 = (seq.T.reshape(nc, Lc, nb, Bb)
            .transpose(0, 2, 1, 3)
            .reshape(nc, nb, 1, N))

    args = (
        seq4,
        embed_W.T,
        ff_w1.T, ff_b1[:, None],
        ff_w2.T, ff_b2[:, None],
        ln_g[:, None], ln_b[:, None],
        gate_w1.T, gate_b1[:, None],
        gate_w2, gate_b2[:, None],
        read_w.T, read_b[:, None],
        out_w.T, out_b[:, None],
    )

    def full_spec(a):
        nd = a.ndim
        return pl.BlockSpec(a.shape, lambda b_, c_, _n=nd: (0,) * _n)

    in_specs = [pl.BlockSpec((1, 1, 1, N), lambda b_, c_: (c_, b_, 0, 0))]
    in_specs += [full_spec(a) for a in args[1:]]

    body = functools.partial(_fused_kernel, nc)

    outT = pl.pallas_call(
        body,
        grid=(nb, nc),
        in_specs=in_specs,
        out_specs=pl.BlockSpec((V, Bb), lambda b_, c_: (0, b_)),
        out_shape=jax.ShapeDtypeStruct((V, B), jnp.float32),
        scratch_shapes=[
            pltpu.VMEM((H, N), jnp.float32),
            pltpu.VMEM((1, N), jnp.float32),
            pltpu.VMEM((1, N), jnp.float32),
            pltpu.VMEM((H, H, Bb), jnp.float32),
        ],
        compiler_params=pltpu.CompilerParams(
            dimension_semantics=("parallel", "arbitrary"),
            vmem_limit_bytes=56 * 1024 * 1024,
        ),
    )(*args)
    return outT.T


# speedup vs baseline: 1.0030x; 1.0030x over previous
import functools

import jax
import jax.numpy as jnp
from jax.experimental import pallas as pl
from jax.experimental.pallas import tpu as pltpu

LN_EPS = 1e-5
DELTA_EPS = 1e-6

Bb = 128
Lc = 128
WIN = 8
JB = 8


def _fused_kernel(nc_total, seq_ref, eWT, w1T, b1c, w2T, b2c, lngc, lnbc,
                  gw1T, gb1c, gw2c, gb2c, rwT, rbc, owT, obc,
                  out_ref, h_s, g_s, rd_s, M_s):
    c = pl.program_id(1)
    V = eWT.shape[1]
    H = h_s.shape[0]
    n = seq_ref.shape[-1]

    seq = seq_ref[0, 0]
    iota_v = jax.lax.broadcasted_iota(jnp.int32, (V, n), 0)
    oh = (iota_v == seq).astype(jnp.float32)
    h0 = jnp.dot(eWT[...], oh, preferred_element_type=jnp.float32)
    ff1 = jnp.maximum(
        jnp.dot(w1T[...], h0, preferred_element_type=jnp.float32) + b1c[...], 0.0)
    h1 = jnp.dot(w2T[...], ff1, preferred_element_type=jnp.float32) + b2c[...] + h0
    mu = jnp.mean(h1, axis=0, keepdims=True)
    d = h1 - mu
    var = jnp.mean(d * d, axis=0, keepdims=True)
    hn = d * jax.lax.rsqrt(var + LN_EPS) * lngc[...] + lnbc[...]
    h_s[...] = hn

    g1 = jnp.maximum(
        jnp.dot(gw1T[...], hn, preferred_element_type=jnp.float32) + gb1c[...], 0.0)
    gp = jnp.sum(g1 * gw2c[...], axis=0, keepdims=True) + gb2c[...]
    gate = jax.nn.sigmoid(gp)
    lastpos = jax.lax.broadcasted_iota(jnp.int32, (1, n), 1) >= (n - Bb)
    is_last_chunk = (c == nc_total - 1)
    gate = jnp.where(jnp.logical_and(is_last_chunk, lastpos), 0.0, gate)
    g_s[...] = gate
    rd_s[...] = 1.0 / (jnp.sum(hn * hn, axis=0, keepdims=True) + DELTA_EPS)

    @pl.when(c == 0)
    def _():
        M_s[...] = jnp.zeros_like(M_s)

    def window(w, carry):
        base = w * (WIN * Bb)
        ks = [h_s[:, pl.ds(base + t * Bb, Bb)] for t in range(WIN)]
        gs = [g_s[:, pl.ds(base + t * Bb, Bb)] for t in range(WIN)]
        rds = [rd_s[:, pl.ds(base + t * Bb, Bb)] for t in range(WIN)]

        acc = [None] * WIN
        for jb in range(0, H, JB):
            blk = M_s[jb:jb + JB]
            for t in range(WIN):
                part = jnp.sum(blk * ks[t][jb:jb + JB, None, :], axis=0)
                acc[t] = part if jb == 0 else acc[t] + part

        us = [None] * WIN
        for t in range(WIN):
            a_t = acc[t]
            for s in range(t):
                d_st = jnp.sum(ks[s] * ks[t], axis=0, keepdims=True)
                a_t = a_t + d_st * us[s]
            us[t] = gs[t] * (ks[t] - a_t * rds[t])

        for jb in range(0, H, JB):
            blk = M_s[jb:jb + JB]
            for t in range(WIN):
                blk = blk + ks[t][jb:jb + JB, None, :] * us[t][None, :, :]
            M_s[jb:jb + JB] = blk
        return carry

    jax.lax.fori_loop(0, Lc // WIN, window, 0)

    @pl.when(c == nc_total - 1)
    def _():
        q = h_s[:, (Lc - 1) * Bb:]
        ctx = jnp.sum(M_s[...] * q[:, None, :], axis=0)
        r = jnp.dot(rwT[...], ctx, preferred_element_type=jnp.float32) + rbc[...]
        out_ref[...] = jnp.dot(owT[...], r, preferred_element_type=jnp.float32) + obc[...]


def kernel(seq, embed_W, ff_w1, ff_b1, ff_w2, ff_b2, ln_g, ln_b,
           gate_w1, gate_b1, gate_w2, gate_b2, read_w, read_b, out_w, out_b):
    B, L = seq.shape
    V, H = embed_W.shape
    nb, nc = B // Bb, L // Lc
    N = Lc * Bb

    seq4 = (seq.T.reshape(nc, Lc, nb, Bb)
            .transpose(0, 2, 1, 3)
            .reshape(nc, nb, 1, N))

    args = (
        seq4,
        embed_W.T,
        ff_w1.T, ff_b1[:, None],
        ff_w2.T, ff_b2[:, None],
        ln_g[:, None], ln_b[:, None],
        gate_w1.T, gate_b1[:, None],
        gate_w2, gate_b2[:, None],
        read_w.T, read_b[:, None],
        out_w.T, out_b[:, None],
    )

    def full_spec(a):
        nd = a.ndim
        return pl.BlockSpec(a.shape, lambda b_, c_, _n=nd: (0,) * _n)

    in_specs = [pl.BlockSpec((1, 1, 1, N), lambda b_, c_: (c_, b_, 0, 0))]
    in_specs += [full_spec(a) for a in args[1:]]

    body = functools.partial(_fused_kernel, nc)

    outT = pl.pallas_call(
        body,
        grid=(nb, nc),
        in_specs=in_specs,
        out_specs=pl.BlockSpec((V, Bb), lambda b_, c_: (0, b_)),
        out_shape=jax.ShapeDtypeStruct((V, B), jnp.float32),
        scratch_shapes=[
            pltpu.VMEM((H, N), jnp.float32),
            pltpu.VMEM((1, N), jnp.float32),
            pltpu.VMEM((1, N), jnp.float32),
            pltpu.VMEM((H, H, Bb), jnp.float32),
        ],
        compiler_params=pltpu.CompilerParams(
            dimension_semantics=("parallel", "arbitrary"),
            vmem_limit_bytes=56 * 1024 * 1024,
        ),
    )(*args)
    return outT.T
